# baseline (speedup 1.0000x reference)
import jax
import jax.numpy as jnp
from jax.experimental import pallas as pl
from jax.experimental.pallas import tpu as pltpu

B = 64
D = 128
M_TOTAL = 100000
INV_TEMP = 1.0 / 0.07
LOG2E = 1.4426950408889634
LN2 = 0.6931471805599453
NSTREAM = 8
CHUNK = 1568
NUM_BLOCKS = 8
NEG = -1e30


def _loss_kernel(x_ref, *refs):
    f_refs = refs[:NSTREAM]
    (fany_ref, cams_ref, camb_ref, idx_ref, out_ref,
     xs_ref, g_ref, m_ref, s_ref, sem) = refs[NSTREAM:]
    j = pl.program_id(0)

    @pl.when(j == 0)
    def _init():
        m_ref[...] = jnp.full((B, NSTREAM), NEG, jnp.float32)
        s_ref[...] = jnp.zeros((B, NSTREAM), jnp.float32)
        xs_ref[...] = x_ref[...] * (INV_TEMP * LOG2E)
        for i in range(B):
            pltpu.make_async_copy(
                fany_ref.at[pl.ds(idx_ref[i], 1), :],
                g_ref.at[pl.ds(i, 1), :], sem).start()

    xs = xs_ref[...]
    camb = camb_ref[...]

    for p, f_ref in enumerate(f_refs):
        logits = jax.lax.dot_general(
            xs, f_ref[...], (((1,), (1,)), ((), ())),
            preferred_element_type=jnp.float32)

        cols = ((NSTREAM * j + p) * CHUNK
                + jax.lax.broadcasted_iota(jnp.int32, (1, CHUNK), 1))
        cams = jnp.where(cols < M_TOTAL,
                         cams_ref[:, pl.ds(p * CHUNK, CHUNK)], -1)
        ml = jnp.where(camb == cams, logits, NEG)

        m_old = m_ref[:, p:p + 1]
        m_new = jnp.maximum(m_old, jnp.max(ml, axis=1, keepdims=True))
        s_ref[:, p:p + 1] = s_ref[:, p:p + 1] * jnp.exp2(m_old - m_new) \
            + jnp.sum(jnp.exp2(ml - m_new), axis=1, keepdims=True)
        m_ref[:, p:p + 1] = m_new

    @pl.when(j == NUM_BLOCKS - 1)
    def _fin():
        for i in range(B):
            pltpu.make_async_copy(
                fany_ref.at[pl.ds(idx_ref[i], 1), :],
                g_ref.at[pl.ds(i, 1), :], sem).wait()
        t = jnp.sum(xs * g_ref[...], axis=1, keepdims=True)
        m_all = m_ref[...]
        m_fin = jnp.max(m_all, axis=1, keepdims=True)
        s_fin = jnp.sum(s_ref[...] * jnp.exp2(m_all - m_fin),
                        axis=1, keepdims=True)
        lse = m_fin + jnp.log2(s_fin)
        out_ref[...] = jnp.sum((lse - t) * (LN2 / B), axis=(0, 1),
                               keepdims=True)


def _f_spec(p):
    return pl.BlockSpec((CHUNK, D), lambda j, p=p: (NSTREAM * j + p, 0))


@jax.jit
def kernel(inputs_features, features, indices, camids_batch, camids):
    camids2 = camids.reshape(1, M_TOTAL)
    camb2 = camids_batch.reshape(B, 1)

    out = pl.pallas_call(
        _loss_kernel,
        grid=(NUM_BLOCKS,),
        in_specs=[pl.BlockSpec((B, D), lambda j: (0, 0))]
        + [_f_spec(p) for p in range(NSTREAM)]
        + [
            pl.BlockSpec(memory_space=pl.ANY),
            pl.BlockSpec((1, NSTREAM * CHUNK), lambda j: (0, j)),
            pl.BlockSpec((B, 1), lambda j: (0, 0)),
            pl.BlockSpec(memory_space=pltpu.SMEM),
        ],
        out_specs=pl.BlockSpec((1, 1), lambda j: (0, 0)),
        out_shape=jax.ShapeDtypeStruct((1, 1), jnp.float32),
        scratch_shapes=[
            pltpu.VMEM((B, D), jnp.float32),
            pltpu.VMEM((B, D), jnp.float32),
            pltpu.VMEM((B, NSTREAM), jnp.float32),
            pltpu.VMEM((B, NSTREAM), jnp.float32),
            pltpu.SemaphoreType.DMA,
        ],
        compiler_params=pltpu.CompilerParams(
            dimension_semantics=("arbitrary",)),
    )(inputs_features, *([features] * NSTREAM), features, camids2, camb2,
      indices)
    return out[0, 0]


# speedup vs baseline: 1.0595x; 1.0595x over previous
import jax
import jax.numpy as jnp
from jax.experimental import pallas as pl
from jax.experimental.pallas import tpu as pltpu

B = 64
D = 128
M_TOTAL = 100000
INV_TEMP = 1.0 / 0.07
LOG2E = 1.4426950408889634
LN2 = 0.6931471805599453
NSTREAM = 16
CHUNK = 1568
NUM_BLOCKS = 4
NEG = -1e30


def _loss_kernel(x_ref, *refs):
    f_refs = refs[:NSTREAM]
    (fany_ref, cams_ref, camb_ref, idx_ref, out_ref,
     xs_ref, g_ref, m_ref, s_ref, sem) = refs[NSTREAM:]
    j = pl.program_id(0)

    @pl.when(j == 0)
    def _init():
        m_ref[...] = jnp.full((B, NSTREAM), NEG, jnp.float32)
        s_ref[...] = jnp.zeros((B, NSTREAM), jnp.float32)
        xs_ref[...] = x_ref[...] * (INV_TEMP * LOG2E)
        for i in range(B):
            pltpu.make_async_copy(
                fany_ref.at[pl.ds(idx_ref[i], 1), :],
                g_ref.at[pl.ds(i, 1), :], sem).start()

    xs = xs_ref[...]
    camb = camb_ref[...]

    for p, f_ref in enumerate(f_refs):
        logits = jax.lax.dot_general(
            xs, f_ref[...], (((1,), (1,)), ((), ())),
            preferred_element_type=jnp.float32)

        cols = ((NSTREAM * j + p) * CHUNK
                + jax.lax.broadcasted_iota(jnp.int32, (1, CHUNK), 1))
        cams = jnp.where(cols < M_TOTAL,
                         cams_ref[:, pl.ds(p * CHUNK, CHUNK)], -1)
        ml = jnp.where(camb == cams, logits, NEG)

        m_old = m_ref[:, p:p + 1]
        m_new = jnp.maximum(m_old, jnp.max(ml, axis=1, keepdims=True))
        s_ref[:, p:p + 1] = s_ref[:, p:p + 1] * jnp.exp2(m_old - m_new) \
            + jnp.sum(jnp.exp2(ml - m_new), axis=1, keepdims=True)
        m_ref[:, p:p + 1] = m_new

    @pl.when(j == NUM_BLOCKS - 1)
    def _fin():
        for i in range(B):
            pltpu.make_async_copy(
                fany_ref.at[pl.ds(idx_ref[i], 1), :],
                g_ref.at[pl.ds(i, 1), :], sem).wait()
        t = jnp.sum(xs * g_ref[...], axis=1, keepdims=True)
        m_all = m_ref[...]
        m_fin = jnp.max(m_all, axis=1, keepdims=True)
        s_fin = jnp.sum(s_ref[...] * jnp.exp2(m_all - m_fin),
                        axis=1, keepdims=True)
        lse = m_fin + jnp.log2(s_fin)
        out_ref[...] = jnp.sum((lse - t) * (LN2 / B), axis=(0, 1),
                               keepdims=True)


def _f_spec(p):
    return pl.BlockSpec((CHUNK, D), lambda j, p=p: (NSTREAM * j + p, 0))


@jax.jit
def kernel(inputs_features, features, indices, camids_batch, camids):
    camids2 = camids.reshape(1, M_TOTAL)
    camb2 = camids_batch.reshape(B, 1)

    out = pl.pallas_call(
        _loss_kernel,
        grid=(NUM_BLOCKS,),
        in_specs=[pl.BlockSpec((B, D), lambda j: (0, 0))]
        + [_f_spec(p) for p in range(NSTREAM)]
        + [
            pl.BlockSpec(memory_space=pl.ANY),
            pl.BlockSpec((1, NSTREAM * CHUNK), lambda j: (0, j)),
            pl.BlockSpec((B, 1), lambda j: (0, 0)),
            pl.BlockSpec(memory_space=pltpu.SMEM),
        ],
        out_specs=pl.BlockSpec((1, 1), lambda j: (0, 0)),
        out_shape=jax.ShapeDtypeStruct((1, 1), jnp.float32),
        scratch_shapes=[
            pltpu.VMEM((B, D), jnp.float32),
            pltpu.VMEM((B, D), jnp.float32),
            pltpu.VMEM((B, NSTREAM), jnp.float32),
            pltpu.VMEM((B, NSTREAM), jnp.float32),
            pltpu.SemaphoreType.DMA,
        ],
        compiler_params=pltpu.CompilerParams(
            dimension_semantics=("arbitrary",)),
    )(inputs_features, *([features] * NSTREAM), features, camids2, camb2,
      indices)
    return out[0, 0]
